# baseline (speedup 1.0000x reference)
import numpy as np
import jax
import jax.numpy as jnp
from jax.experimental import pallas as pl
from jax.experimental.pallas import tpu as pltpu


def _conv_stats_kernel(x_ref, g_ref, w_ref, conv_ref, stats_ref):
    v = jnp.maximum(x_ref[0], 0.0)
    p = jnp.dot(v, g_ref[...], preferred_element_type=jnp.float32)
    s = p.shape[1] // 2
    pv = jnp.concatenate([p[:, :s], p[:, s:]], axis=0)
    y = jnp.dot(w_ref[...], pv, preferred_element_type=jnp.float32)
    conv_ref[0] = y.astype(conv_ref.dtype)
    ycat = jnp.concatenate([y, y * y], axis=0)
    ones = jnp.ones((8, s), jnp.float32)
    stats_ref[0] = jax.lax.dot_general(
        ones, ycat, dimension_numbers=(((1,), (1,)), ((), ())),
        preferred_element_type=jnp.float32)


def _bn_apply_kernel(conv_ref, stats_ref, gamma_ref, beta_ref, o_ref,
                     *, count, eps):
    c = gamma_ref.shape[1]
    s = o_ref.shape[2]
    tot = jnp.sum(stats_ref[...], axis=0)
    row = tot[0:1, :]
    inv_n = 1.0 / count
    mean = row[:, :c] * inv_n
    var = row[:, c:] * inv_n - mean * mean
    scale = gamma_ref[...] * jax.lax.rsqrt(var + eps)
    shift = beta_ref[...] - mean * scale
    ones = jnp.ones((1, s), jnp.float32)
    dn = (((0,), (0,)), ((), ()))
    scale_t = jax.lax.dot_general(scale, ones, dimension_numbers=dn,
                                  preferred_element_type=jnp.float32)
    shift_t = jax.lax.dot_general(shift, ones, dimension_numbers=dn,
                                  preferred_element_type=jnp.float32)
    y = conv_ref[...].astype(jnp.float32)
    o_ref[...] = y * scale_t[None] + shift_t[None]


def kernel(x_nchw, w1, b1, w2, b2, gamma, beta, *, eps=1e-5):
    n, cin, h, w = x_nchw.shape
    half = w1.shape[0]
    cout = 2 * half
    oh, ow = h // 2, w // 2
    s = oh * ow
    hw = h * w
    rows = n * s

    x_flat = x_nchw.astype(jnp.float32).reshape(n, cin, hw)

    jj = np.arange(s)
    r_, q_ = jj // ow, jj % ow
    g_np = np.zeros((hw, 2 * s), np.float32)
    g_np[(2 * r_) * w + 2 * q_, jj] = 1.0
    g_np[(2 * r_ + 1) * w + (2 * q_ + 1), s + jj] = 1.0
    g = jnp.asarray(g_np)

    w_bd = jnp.concatenate(
        [jnp.concatenate([w1.astype(jnp.float32),
                          jnp.zeros((half, cin), jnp.float32)], axis=1),
         jnp.concatenate([jnp.zeros((half, cin), jnp.float32),
                          w2.astype(jnp.float32)], axis=1)], axis=0)
    del b1, b2
    g_row = gamma.astype(jnp.float32).reshape(1, cout)
    beta_row = beta.astype(jnp.float32).reshape(1, cout)

    cparams = pltpu.CompilerParams(
        dimension_semantics=("parallel",),
        vmem_limit_bytes=64 * 1024 * 1024,
    )

    conv, stats = pl.pallas_call(
        _conv_stats_kernel,
        grid=(n,),
        in_specs=[pl.BlockSpec((1, cin, hw), lambda i: (i, 0, 0)),
                  pl.BlockSpec((hw, 2 * s), lambda i: (0, 0)),
                  pl.BlockSpec((cout, 2 * cin), lambda i: (0, 0))],
        out_specs=(pl.BlockSpec((1, cout, s), lambda i: (i, 0, 0)),
                   pl.BlockSpec((1, 8, 2 * cout), lambda i: (i, 0, 0))),
        out_shape=(jax.ShapeDtypeStruct((n, cout, s), jnp.bfloat16),
                   jax.ShapeDtypeStruct((n, 8, 2 * cout), jnp.float32)),
        compiler_params=cparams,
        cost_estimate=pl.CostEstimate(
            flops=2 * rows * (2 * cin) * cout + 2 * n * cin * hw * 2 * s,
            transcendentals=0,
            bytes_accessed=4 * (n * cin * hw + hw * 2 * s)
            + 2 * n * cout * s + 4 * n * 8 * 2 * cout),
    )(x_flat, g, w_bd)

    nb = 4
    while n % nb:
        nb -= 1
    import functools
    out = pl.pallas_call(
        functools.partial(_bn_apply_kernel, count=float(rows), eps=eps),
        grid=(n // nb,),
        in_specs=[pl.BlockSpec((nb, cout, s), lambda i: (i, 0, 0)),
                  pl.BlockSpec((n, 8, 2 * cout), lambda i: (0, 0, 0)),
                  pl.BlockSpec((1, cout), lambda i: (0, 0)),
                  pl.BlockSpec((1, cout), lambda i: (0, 0))],
        out_specs=pl.BlockSpec((nb, cout, s), lambda i: (i, 0, 0)),
        out_shape=jax.ShapeDtypeStruct((n, cout, s), jnp.float32),
        compiler_params=cparams,
        cost_estimate=pl.CostEstimate(
            flops=2 * rows * cout + 16 * n * cout,
            transcendentals=cout,
            bytes_accessed=2 * n * cout * s + 4 * n * cout * s
            + 4 * n * 8 * 2 * cout),
    )(conv, stats, g_row, beta_row)

    return out.reshape(n, cout, oh, ow)
